# initial kernel scaffold (unmeasured)
import functools

import jax
import jax.numpy as jnp
from jax import lax
from jax.experimental import pallas as pl
from jax.experimental.pallas import tpu as pltpu

M = 8192
D = 4096
HALF = M // 2
QTR = HALF // 2
CHUNK = 512
N_CHUNKS = QTR // CHUNK


def kernel(partial, gamma):
    p2d = partial.reshape(M, D)

    def body(p_ref, g_ref, out_ref, rbuf_ref,
             a_ref, b_ref, o_ref,
             send_x, recv_x, send_y, recv_y, local_sems):
        my_x = lax.axis_index("x")
        my_y = lax.axis_index("y")
        nbr_x = (1 - my_x, my_y)
        nbr_y = (my_x, 1 - my_y)

        barrier = pltpu.get_barrier_semaphore()
        for nbr in (nbr_x, nbr_y):
            pl.semaphore_signal(barrier, inc=1, device_id=nbr,
                                device_id_type=pl.DeviceIdType.MESH)
        pl.semaphore_wait(barrier, 2)

        my_q0 = my_x * HALF + my_y * QTR
        nbr_q0 = (1 - my_x) * HALF + my_y * QTR

        rdma_x = pltpu.make_async_remote_copy(
            src_ref=p_ref.at[pl.ds(nbr_q0, QTR), :],
            dst_ref=rbuf_ref,
            send_sem=send_x,
            recv_sem=recv_x,
            device_id=nbr_x,
            device_id_type=pl.DeviceIdType.MESH,
        )
        rdma_x.start()
        rdma_x.wait()

        g = g_ref[...]
        out_row0 = my_y * QTR
        for c in range(N_CHUNKS):
            row = c * CHUNK
            cp_a = pltpu.make_async_copy(
                p_ref.at[pl.ds(my_q0 + row, CHUNK), :], a_ref,
                local_sems.at[0])
            cp_b = pltpu.make_async_copy(
                rbuf_ref.at[pl.ds(row, CHUNK), :], b_ref, local_sems.at[1])
            cp_a.start()
            cp_b.start()
            cp_a.wait()
            cp_b.wait()

            y = a_ref[...] + b_ref[...]
            rms = jnp.sqrt(jnp.mean(y * y, axis=-1, keepdims=True) + 1e-6)
            o_ref[...] = y / rms * g

            store = pltpu.make_async_copy(
                o_ref, out_ref.at[pl.ds(out_row0 + row, CHUNK), :],
                local_sems.at[0])
            rdma_y = pltpu.make_async_remote_copy(
                src_ref=o_ref,
                dst_ref=out_ref.at[pl.ds(out_row0 + row, CHUNK), :],
                send_sem=send_y.at[c],
                recv_sem=recv_y.at[c],
                device_id=nbr_y,
                device_id_type=pl.DeviceIdType.MESH,
            )
            store.start()
            rdma_y.start()
            store.wait()
            rdma_y.wait()

        @functools.partial(pl.run_scoped, exit_sem=pltpu.SemaphoreType.REGULAR)
        def _(exit_sem):
            for nbr in (nbr_x, nbr_y):
                pl.semaphore_signal(exit_sem, inc=1, device_id=nbr,
                                    device_id_type=pl.DeviceIdType.MESH)
            pl.semaphore_wait(exit_sem, 2)

    out, _ = pl.pallas_call(
        body,
        out_shape=(
            jax.ShapeDtypeStruct((HALF, D), jnp.float32),
            jax.ShapeDtypeStruct((QTR, D), jnp.float32),
        ),
        in_specs=[
            pl.BlockSpec(memory_space=pltpu.MemorySpace.ANY),
            pl.BlockSpec(memory_space=pltpu.MemorySpace.VMEM),
        ],
        out_specs=(
            pl.BlockSpec(memory_space=pltpu.MemorySpace.ANY),
            pl.BlockSpec(memory_space=pltpu.MemorySpace.ANY),
        ),
        scratch_shapes=[
            pltpu.VMEM((CHUNK, D), jnp.float32),
            pltpu.VMEM((CHUNK, D), jnp.float32),
            pltpu.VMEM((CHUNK, D), jnp.float32),
            pltpu.SemaphoreType.DMA,
            pltpu.SemaphoreType.DMA,
            pltpu.SemaphoreType.DMA((N_CHUNKS,)),
            pltpu.SemaphoreType.DMA((N_CHUNKS,)),
            pltpu.SemaphoreType.DMA((2,)),
        ],
        compiler_params=pltpu.CompilerParams(collective_id=0),
    )(p2d, gamma)
    return out


# baseline (device time: 808588 ns/iter reference)
import functools

import jax
import jax.numpy as jnp
from jax import lax
from jax.experimental import pallas as pl
from jax.experimental.pallas import tpu as pltpu

M = 8192
D = 4096
HALF = M // 2
QTR = HALF // 2
CHUNK = 512
N_CHUNKS = QTR // CHUNK


def kernel(partial, gamma):
    p2d = partial.reshape(M, D)

    def body(p_ref, g_ref, out_ref, rbuf_ref,
             a_ref, b_ref, o_ref,
             send_x, recv_x, send_y, recv_y, local_sems):
        my_x = lax.axis_index("x")
        my_y = lax.axis_index("y")
        nbr_x = (1 - my_x, my_y)
        nbr_y = (my_x, 1 - my_y)

        barrier = pltpu.get_barrier_semaphore()
        for nbr in (nbr_x, nbr_y):
            pl.semaphore_signal(barrier, inc=1, device_id=nbr,
                                device_id_type=pl.DeviceIdType.MESH)
        pl.semaphore_wait(barrier, 2)

        my_q0 = my_x * HALF + my_y * QTR
        nbr_q0 = (1 - my_x) * HALF + my_y * QTR

        rdma_x = pltpu.make_async_remote_copy(
            src_ref=p_ref.at[pl.ds(nbr_q0, QTR), :],
            dst_ref=rbuf_ref,
            send_sem=send_x,
            recv_sem=recv_x,
            device_id=nbr_x,
            device_id_type=pl.DeviceIdType.MESH,
        )
        rdma_x.start()
        rdma_x.wait()

        g = g_ref[...]
        out_row0 = my_y * QTR
        for c in range(N_CHUNKS):
            row = c * CHUNK
            cp_a = pltpu.make_async_copy(
                p_ref.at[pl.ds(my_q0 + row, CHUNK), :], a_ref,
                local_sems.at[0])
            cp_b = pltpu.make_async_copy(
                rbuf_ref.at[pl.ds(row, CHUNK), :], b_ref, local_sems.at[1])
            cp_a.start()
            cp_b.start()
            cp_a.wait()
            cp_b.wait()

            y = a_ref[...] + b_ref[...]
            rms = jnp.sqrt(jnp.mean(y * y, axis=-1, keepdims=True) + 1e-6)
            o_ref[...] = y / rms * g

            store = pltpu.make_async_copy(
                o_ref, out_ref.at[pl.ds(out_row0 + row, CHUNK), :],
                local_sems.at[0])
            rdma_y = pltpu.make_async_remote_copy(
                src_ref=o_ref,
                dst_ref=out_ref.at[pl.ds(out_row0 + row, CHUNK), :],
                send_sem=send_y.at[c],
                recv_sem=recv_y.at[c],
                device_id=nbr_y,
                device_id_type=pl.DeviceIdType.MESH,
            )
            store.start()
            rdma_y.start()
            store.wait()
            rdma_y.wait()

        @functools.partial(pl.run_scoped, exit_sem=pltpu.SemaphoreType.REGULAR)
        def _(exit_sem):
            for nbr in (nbr_x, nbr_y):
                pl.semaphore_signal(exit_sem, inc=1, device_id=nbr,
                                    device_id_type=pl.DeviceIdType.MESH)
            pl.semaphore_wait(exit_sem, 2)

    out, _ = pl.pallas_call(
        body,
        out_shape=(
            jax.ShapeDtypeStruct((HALF, D), jnp.float32),
            jax.ShapeDtypeStruct((QTR, D), jnp.float32),
        ),
        in_specs=[
            pl.BlockSpec(memory_space=pl.ANY),
            pl.BlockSpec(memory_space=pltpu.MemorySpace.VMEM),
        ],
        out_specs=(
            pl.BlockSpec(memory_space=pl.ANY),
            pl.BlockSpec(memory_space=pl.ANY),
        ),
        scratch_shapes=[
            pltpu.VMEM((CHUNK, D), jnp.float32),
            pltpu.VMEM((CHUNK, D), jnp.float32),
            pltpu.VMEM((CHUNK, D), jnp.float32),
            pltpu.SemaphoreType.DMA,
            pltpu.SemaphoreType.DMA,
            pltpu.SemaphoreType.DMA((N_CHUNKS,)),
            pltpu.SemaphoreType.DMA((N_CHUNKS,)),
            pltpu.SemaphoreType.DMA((2,)),
        ],
        compiler_params=pltpu.CompilerParams(
            collective_id=0, vmem_limit_bytes=64 * 1024 * 1024),
    )(p2d, gamma)
    return out


# device time: 458715 ns/iter; 1.7627x vs baseline; 1.7627x over previous
import functools

import jax
import jax.numpy as jnp
from jax import lax
from jax.experimental import pallas as pl
from jax.experimental.pallas import tpu as pltpu

M = 8192
D = 4096
HALF = M // 2
QTR = HALF // 2
CHUNK = 256
N_CHUNKS = QTR // CHUNK


def kernel(partial, gamma):
    p2d = partial.reshape(M, D)

    def body(p_ref, g_ref, out_ref, rbuf_ref, a_bufs, o_bufs,
             send_x, recv_x, send_y, recv_y, a_sems, store_sems):
        my_x = lax.axis_index("x")
        my_y = lax.axis_index("y")
        nbr_x = (1 - my_x, my_y)
        nbr_y = (my_x, 1 - my_y)

        barrier = pltpu.get_barrier_semaphore()
        for nbr in (nbr_x, nbr_y):
            pl.semaphore_signal(barrier, inc=1, device_id=nbr,
                                device_id_type=pl.DeviceIdType.MESH)
        pl.semaphore_wait(barrier, 2)

        my_q0 = my_x * HALF + my_y * QTR
        nbr_q0 = (1 - my_x) * HALF + my_y * QTR

        rdx = []
        for c in range(N_CHUNKS):
            r = pltpu.make_async_remote_copy(
                src_ref=p_ref.at[pl.ds(nbr_q0 + c * CHUNK, CHUNK), :],
                dst_ref=rbuf_ref.at[pl.ds(c * CHUNK, CHUNK), :],
                send_sem=send_x.at[c],
                recv_sem=recv_x.at[c],
                device_id=nbr_x,
                device_id_type=pl.DeviceIdType.MESH,
            )
            r.start()
            rdx.append(r)

        acp = {}
        for c in range(min(2, N_CHUNKS)):
            acp[c] = pltpu.make_async_copy(
                p_ref.at[pl.ds(my_q0 + c * CHUNK, CHUNK), :],
                a_bufs.at[c % 2], a_sems.at[c % 2])
            acp[c].start()

        g = g_ref[...]
        out_row0 = my_y * QTR
        stores = {}
        rdy = {}
        for c in range(N_CHUNKS):
            s = c % 2
            if c >= 2:
                stores[c - 2].wait()
                rdy[c - 2].wait_send()
            acp[c].wait()
            rdx[c].wait()

            yv = a_bufs[s] + rbuf_ref[pl.ds(c * CHUNK, CHUNK), :]
            rms = jnp.sqrt(jnp.mean(yv * yv, axis=-1, keepdims=True) + 1e-6)
            o_bufs[s] = yv / rms * g

            stores[c] = pltpu.make_async_copy(
                o_bufs.at[s], out_ref.at[pl.ds(out_row0 + c * CHUNK, CHUNK), :],
                store_sems.at[s])
            stores[c].start()
            rdy[c] = pltpu.make_async_remote_copy(
                src_ref=o_bufs.at[s],
                dst_ref=out_ref.at[pl.ds(out_row0 + c * CHUNK, CHUNK), :],
                send_sem=send_y.at[c],
                recv_sem=recv_y.at[c],
                device_id=nbr_y,
                device_id_type=pl.DeviceIdType.MESH,
            )
            rdy[c].start()

            if c + 2 < N_CHUNKS:
                acp[c + 2] = pltpu.make_async_copy(
                    p_ref.at[pl.ds(my_q0 + (c + 2) * CHUNK, CHUNK), :],
                    a_bufs.at[s], a_sems.at[s])
                acp[c + 2].start()

        for c in range(max(0, N_CHUNKS - 2), N_CHUNKS):
            stores[c].wait()
            rdy[c].wait_send()
        for c in range(N_CHUNKS):
            rdy[c].wait_recv()

        @functools.partial(pl.run_scoped, exit_sem=pltpu.SemaphoreType.REGULAR)
        def _(exit_sem):
            for nbr in (nbr_x, nbr_y):
                pl.semaphore_signal(exit_sem, inc=1, device_id=nbr,
                                    device_id_type=pl.DeviceIdType.MESH)
            pl.semaphore_wait(exit_sem, 2)

    return pl.pallas_call(
        body,
        out_shape=jax.ShapeDtypeStruct((HALF, D), jnp.float32),
        in_specs=[
            pl.BlockSpec(memory_space=pl.ANY),
            pl.BlockSpec(memory_space=pltpu.MemorySpace.VMEM),
        ],
        out_specs=pl.BlockSpec(memory_space=pl.ANY),
        scratch_shapes=[
            pltpu.VMEM((QTR, D), jnp.float32),
            pltpu.VMEM((2, CHUNK, D), jnp.float32),
            pltpu.VMEM((2, CHUNK, D), jnp.float32),
            pltpu.SemaphoreType.DMA((N_CHUNKS,)),
            pltpu.SemaphoreType.DMA((N_CHUNKS,)),
            pltpu.SemaphoreType.DMA((N_CHUNKS,)),
            pltpu.SemaphoreType.DMA((N_CHUNKS,)),
            pltpu.SemaphoreType.DMA((2,)),
            pltpu.SemaphoreType.DMA((2,)),
        ],
        compiler_params=pltpu.CompilerParams(
            collective_id=0, vmem_limit_bytes=64 * 1024 * 1024),
    )(p2d, gamma)
